# baseline (speedup 1.0000x reference)
import functools

import jax
import jax.numpy as jnp
from jax import lax
from jax.experimental import pallas as pl
from jax.experimental.pallas import tpu as pltpu
from jax.experimental.pallas import tpu_sc as plsc

N_NODES = 10000
N_EDGES = 320000
D_FEAT = 128
D_HALF = D_FEAT // 2

NUM_CORES = 2
NUM_TILES = 16

EDGES_PER_TILE = N_EDGES // NUM_TILES
CHUNK = 128
NUM_CHUNKS = -(-EDGES_PER_TILE // CHUNK)
NUM_CHUNKS_EVEN = NUM_CHUNKS + (NUM_CHUNKS % 2)
PAD_EDGES = NUM_CHUNKS_EVEN * CHUNK - EDGES_PER_TILE
DUMMY_ROW = N_NODES
ROWS_PER_TILE = 632
ACC_ROWS = NUM_TILES * ROWS_PER_TILE
LAST_ROWS = N_NODES - (NUM_TILES - 1) * ROWS_PER_TILE


def _gin_sc_call(nodex, srcx, dstp, n0s):
    mesh = plsc.VectorSubcoreMesh(core_axis_name="c", subcore_axis_name="s")

    @functools.partial(
        pl.kernel,
        out_type=jax.ShapeDtypeStruct((N_NODES, D_FEAT), jnp.float32),
        mesh=mesh,
        scratch_types=[
            pltpu.VMEM((NUM_CHUNKS_EVEN + 1, CHUNK), jnp.int32),
            pltpu.VMEM((NUM_CHUNKS_EVEN, CHUNK), jnp.int32),
            pltpu.VMEM((CHUNK, D_HALF), jnp.float32),
            pltpu.VMEM((CHUNK, D_HALF), jnp.float32),
            pltpu.VMEM_SHARED((ACC_ROWS, D_HALF), jnp.float32),
            pltpu.SemaphoreType.DMA,
            pltpu.SemaphoreType.DMA,
        ],
        compiler_params=pltpu.CompilerParams(use_tc_tiling_on_sc=False),
    )
    def k(nodex_hbm, srcx_hbm, dstp_hbm, n0s_hbm, out_hbm,
          src_v, dst_v, buf_a, buf_b, acc, sem_a, sem_b):
        c = lax.axis_index("c")
        s = lax.axis_index("s")
        w = c * NUM_TILES + s
        pltpu.sync_copy(srcx_hbm.at[w], src_v)
        pltpu.sync_copy(dstp_hbm.at[s], dst_v)
        row0 = s * ROWS_PER_TILE

        @pl.when(s < NUM_TILES - 1)
        def _():
            pltpu.sync_copy(
                n0s_hbm.at[pl.ds(row0, ROWS_PER_TILE), pl.ds(c * D_HALF, D_HALF)],
                acc.at[pl.ds(row0, ROWS_PER_TILE)])

        @pl.when(s == NUM_TILES - 1)
        def _():
            pltpu.sync_copy(
                n0s_hbm.at[pl.ds(row0, LAST_ROWS), pl.ds(c * D_HALF, D_HALF)],
                acc.at[pl.ds(row0, LAST_ROWS)])

        plsc.subcore_barrier()

        def start(chunk, buf, sem):
            pltpu.make_async_copy(nodex_hbm.at[src_v.at[chunk]], buf, sem).start()

        def wait(chunk, buf, sem):
            pltpu.make_async_copy(nodex_hbm.at[src_v.at[chunk]], buf, sem).wait()

        start(0, buf_a, sem_a)

        def body(i, carry):
            c0 = 2 * i
            start(c0 + 1, buf_b, sem_b)
            wait(c0, buf_a, sem_a)
            pltpu.sync_copy(buf_a, acc.at[dst_v.at[c0]], add=True)
            start(c0 + 2, buf_a, sem_a)
            wait(c0 + 1, buf_b, sem_b)
            pltpu.sync_copy(buf_b, acc.at[dst_v.at[c0 + 1]], add=True)
            return carry

        lax.fori_loop(0, NUM_CHUNKS_EVEN // 2, body, 0)
        wait(NUM_CHUNKS_EVEN, buf_a, sem_a)
        plsc.subcore_barrier()

        @pl.when(s < NUM_TILES - 1)
        def _():
            pltpu.sync_copy(
                acc.at[pl.ds(row0, ROWS_PER_TILE)],
                out_hbm.at[pl.ds(row0, ROWS_PER_TILE), pl.ds(c * D_HALF, D_HALF)])

        @pl.when(s == NUM_TILES - 1)
        def _():
            pltpu.sync_copy(
                acc.at[pl.ds(row0, LAST_ROWS)],
                out_hbm.at[pl.ds(row0, LAST_ROWS), pl.ds(c * D_HALF, D_HALF)])

    return k(nodex, srcx, dstp, n0s)


def kernel(node, edge_index, node_0, eps_k):
    nodex = node.reshape(NUM_CORES * N_NODES, D_HALF)
    src2 = (edge_index[1] * 2).reshape(NUM_TILES, EDGES_PER_TILE)
    src2 = jnp.pad(src2, ((0, 0), (0, PAD_EDGES + CHUNK)))
    src2 = src2.reshape(NUM_TILES, NUM_CHUNKS_EVEN + 1, CHUNK)
    srcx = jnp.stack([src2, src2 + 1]).reshape(
        NUM_CORES * NUM_TILES, NUM_CHUNKS_EVEN + 1, CHUNK)
    dst_t = edge_index[0].reshape(NUM_TILES, EDGES_PER_TILE)
    if PAD_EDGES:
        pad_rows = DUMMY_ROW + (jnp.arange(PAD_EDGES, dtype=jnp.int32) % 8)
        dst_t = jnp.concatenate(
            [dst_t, jnp.broadcast_to(pad_rows, (NUM_TILES, PAD_EDGES))], axis=1)
    dstp = dst_t.reshape(NUM_TILES, NUM_CHUNKS_EVEN, CHUNK)
    n0s = (jnp.float32(1.0) + eps_k) * node_0
    return _gin_sc_call(nodex, srcx, dstp, n0s)


# speedup vs baseline: 1.5422x; 1.5422x over previous
import functools

import jax
import jax.numpy as jnp
from jax import lax
from jax.experimental import pallas as pl
from jax.experimental.pallas import tpu as pltpu
from jax.experimental.pallas import tpu_sc as plsc

N_NODES = 10000
N_EDGES = 320000
D_FEAT = 128
D_HALF = D_FEAT // 2

NUM_CORES = 2
NUM_TILES = 16

EDGES_PER_TILE = N_EDGES // NUM_TILES
CHUNK = 80
NUM_CHUNKS = -(-EDGES_PER_TILE // CHUNK)
NUM_CHUNKS_EVEN = NUM_CHUNKS + (NUM_CHUNKS % 2)
PAD_EDGES = NUM_CHUNKS_EVEN * CHUNK - EDGES_PER_TILE
DUMMY_ROW = N_NODES
ROWS_PER_TILE = 632
ACC_ROWS = NUM_TILES * ROWS_PER_TILE
LAST_ROWS = N_NODES - (NUM_TILES - 1) * ROWS_PER_TILE


def _gin_sc_call(nodex, srcx, dstp, n0s):
    mesh = plsc.VectorSubcoreMesh(core_axis_name="c", subcore_axis_name="s")

    @functools.partial(
        pl.kernel,
        out_type=jax.ShapeDtypeStruct((N_NODES, D_FEAT), jnp.float32),
        mesh=mesh,
        scratch_types=[
            pltpu.VMEM((NUM_CHUNKS_EVEN + 1, CHUNK), jnp.int32),
            pltpu.VMEM((NUM_CHUNKS_EVEN, CHUNK), jnp.int32),
            pltpu.VMEM((CHUNK, D_HALF), jnp.float32),
            pltpu.VMEM((CHUNK, D_HALF), jnp.float32),
            pltpu.VMEM_SHARED((ACC_ROWS, D_HALF), jnp.float32),
            pltpu.SemaphoreType.DMA,
            pltpu.SemaphoreType.DMA,
        ],
        compiler_params=pltpu.CompilerParams(use_tc_tiling_on_sc=False),
    )
    def k(nodex_hbm, srcx_hbm, dstp_hbm, n0s_hbm, out_hbm,
          src_v, dst_v, buf_a, buf_b, acc, sem_a, sem_b):
        c = lax.axis_index("c")
        s = lax.axis_index("s")
        w = c * NUM_TILES + s
        pltpu.sync_copy(srcx_hbm.at[w], src_v)
        pltpu.sync_copy(dstp_hbm.at[s], dst_v)
        row0 = s * ROWS_PER_TILE

        @pl.when(s < NUM_TILES - 1)
        def _():
            pltpu.sync_copy(
                n0s_hbm.at[pl.ds(row0, ROWS_PER_TILE), pl.ds(c * D_HALF, D_HALF)],
                acc.at[pl.ds(row0, ROWS_PER_TILE)])

        @pl.when(s == NUM_TILES - 1)
        def _():
            pltpu.sync_copy(
                n0s_hbm.at[pl.ds(row0, LAST_ROWS), pl.ds(c * D_HALF, D_HALF)],
                acc.at[pl.ds(row0, LAST_ROWS)])

        plsc.subcore_barrier()

        def start(chunk, buf, sem):
            pltpu.make_async_copy(nodex_hbm.at[src_v.at[chunk]], buf, sem).start()

        def wait(chunk, buf, sem):
            pltpu.make_async_copy(nodex_hbm.at[src_v.at[chunk]], buf, sem).wait()

        start(0, buf_a, sem_a)

        def body(i, carry):
            c0 = 2 * i
            start(c0 + 1, buf_b, sem_b)
            wait(c0, buf_a, sem_a)
            pltpu.sync_copy(buf_a, acc.at[dst_v.at[c0]], add=True)
            start(c0 + 2, buf_a, sem_a)
            wait(c0 + 1, buf_b, sem_b)
            pltpu.sync_copy(buf_b, acc.at[dst_v.at[c0 + 1]], add=True)
            return carry

        lax.fori_loop(0, NUM_CHUNKS_EVEN // 2, body, 0)
        wait(NUM_CHUNKS_EVEN, buf_a, sem_a)
        plsc.subcore_barrier()

        @pl.when(s < NUM_TILES - 1)
        def _():
            pltpu.sync_copy(
                acc.at[pl.ds(row0, ROWS_PER_TILE)],
                out_hbm.at[pl.ds(row0, ROWS_PER_TILE), pl.ds(c * D_HALF, D_HALF)])

        @pl.when(s == NUM_TILES - 1)
        def _():
            pltpu.sync_copy(
                acc.at[pl.ds(row0, LAST_ROWS)],
                out_hbm.at[pl.ds(row0, LAST_ROWS), pl.ds(c * D_HALF, D_HALF)])

    return k(nodex, srcx, dstp, n0s)


def kernel(node, edge_index, node_0, eps_k):
    nodex = node.reshape(NUM_CORES * N_NODES, D_HALF)
    src2 = (edge_index[1] * 2).reshape(NUM_TILES, EDGES_PER_TILE)
    src2 = jnp.pad(src2, ((0, 0), (0, PAD_EDGES + CHUNK)))
    src2 = src2.reshape(NUM_TILES, NUM_CHUNKS_EVEN + 1, CHUNK)
    srcx = jnp.stack([src2, src2 + 1]).reshape(
        NUM_CORES * NUM_TILES, NUM_CHUNKS_EVEN + 1, CHUNK)
    dst_t = edge_index[0].reshape(NUM_TILES, EDGES_PER_TILE)
    if PAD_EDGES:
        pad_rows = DUMMY_ROW + (jnp.arange(PAD_EDGES, dtype=jnp.int32) % 8)
        dst_t = jnp.concatenate(
            [dst_t, jnp.broadcast_to(pad_rows, (NUM_TILES, PAD_EDGES))], axis=1)
    dstp = dst_t.reshape(NUM_TILES, NUM_CHUNKS_EVEN, CHUNK)
    n0s = (jnp.float32(1.0) + eps_k) * node_0
    return _gin_sc_call(nodex, srcx, dstp, n0s)
